# baseline (speedup 1.0000x reference)
import functools

import jax
import jax.numpy as jnp
import numpy as np
from jax import lax
from jax.experimental import pallas as pl
from jax.experimental.pallas import tpu as pltpu
from jax.experimental.pallas import tpu_sc as plsc

N = 10000
E = 160000
D_IN = 256
D_OUT = 256
HEADS = 4
OUT_CH = 64

NW = 32
ROWS = 320
NPAD = NW * ROWS
AW = 288
CAP = 5760
KB = 32
EB = 4000

BLK = 400


def _p1_body(x_ref, w_ref, b_ref, p_ref, ilv_ref, hbf_ref, hf_ref, auxd_ref,
             aux2_ref):
    xb = x_ref[...]
    h = jnp.dot(xb, w_ref[...], preferred_element_type=jnp.float32)
    h = h + b_ref[...]
    aux = jnp.dot(h, p_ref[...], preferred_element_type=jnp.float32)
    ones = jnp.ones((BLK, 4), jnp.float32)
    z4 = jnp.zeros((BLK, 4), jnp.float32)
    z8 = jnp.zeros((BLK, 8), jnp.float32)
    vale = jnp.concatenate([ones, aux[:, 0:4]], axis=1)
    inter = jnp.dot(vale, ilv_ref[...], preferred_element_type=jnp.float32)
    hbf_ref[...] = jnp.concatenate([h, inter], axis=1).astype(jnp.bfloat16)
    hf_ref[...] = h
    auxd_ref[...] = jnp.concatenate([z4, aux[:, 4:8], z8], axis=1)
    aux2_ref[...] = aux[:, 8:24]


def _phase1(x, W, b2d, Pmat, Ilv):
    grid = (N // BLK,)
    return pl.pallas_call(
        _p1_body,
        grid=grid,
        in_specs=[
            pl.BlockSpec((BLK, D_IN), lambda i: (i, 0)),
            pl.BlockSpec((D_IN, D_OUT), lambda i: (0, 0)),
            pl.BlockSpec((1, D_OUT), lambda i: (0, 0)),
            pl.BlockSpec((D_IN, 32), lambda i: (0, 0)),
            pl.BlockSpec((8, 32), lambda i: (0, 0)),
        ],
        out_specs=[
            pl.BlockSpec((BLK, AW), lambda i: (i, 0)),
            pl.BlockSpec((BLK, D_OUT), lambda i: (i, 0)),
            pl.BlockSpec((BLK, 16), lambda i: (i, 0)),
            pl.BlockSpec((BLK, 16), lambda i: (i, 0)),
        ],
        out_shape=[
            jax.ShapeDtypeStruct((N, AW), jnp.bfloat16),
            jax.ShapeDtypeStruct((N, D_OUT), jnp.float32),
            jax.ShapeDtypeStruct((N, 16), jnp.float32),
            jax.ShapeDtypeStruct((N, 16), jnp.float32),
        ],
    )(x, W, b2d, Pmat, Ilv)


_GD = lax.GatherDimensionNumbers(offset_dims=(), collapsed_slice_dims=(0,),
                                 start_index_map=(0,))


def _rgather(v, idx):
    return lax.gather(v, idx[:, None], _GD, slice_sizes=(1,),
                      mode=lax.GatherScatterMode.PROMISE_IN_BOUNDS)


def _sc_body(edge_hbm, haug_hbm, auxd_hbm, zi_hbm, zf_hbm,
             out_hbm, selsrc, seldst, esrc, edst, h0, h1, a0, a1, wbuf,
             acc, semh0, semh1, sema0, sema1, semes, semed):
    cid = lax.axis_index("c")
    sid = lax.axis_index("s")
    wid = sid * 2 + cid
    base = wid * ROWS
    iota16 = lax.broadcasted_iota(jnp.int32, (16,), 0)

    pltpu.sync_copy(zf_hbm, acc)
    pltpu.sync_copy(zi_hbm, selsrc)
    pltpu.sync_copy(zi_hbm, seldst)

    def blk_body(bi, cnt):
        cps = pltpu.async_copy(edge_hbm.at[0, pl.ds(bi * EB, EB)], esrc,
                               semes)
        cpd = pltpu.async_copy(edge_hbm.at[1, pl.ds(bi * EB, EB)], edst,
                               semed)
        cps.wait()
        cpd.wait()

        def in_body(t, cntv):
            d = edst[pl.ds(t * 16, 16)]
            ld = d - base
            m = (ld >= 0) & (ld < ROWS)
            pcv = plsc.all_reduce_population_count(m)

            @pl.when(pcv[0] > 0)
            def _():
                s = esrc[pl.ds(t * 16, 16)]
                pos = plsc.cumsum(m.astype(jnp.int32))
                idx = jnp.minimum(cntv, CAP - 16) + pos - 1
                plsc.store_scatter(selsrc, [idx], s, mask=m)
                plsc.store_scatter(seldst, [idx], d, mask=m)

            return cntv + pcv

        return plsc.parallel_loop(0, EB // 16, unroll=4, carry=cnt)(in_body)

    cnt0 = jnp.zeros((16,), jnp.int32)
    countv = lax.fori_loop(0, E // EB, blk_body, cnt0)
    count = jnp.minimum(countv[0], CAP)

    def g_issue(b, hb, ab, sh, sa):
        off = jnp.minimum(b * KB, CAP - KB)
        pltpu.async_copy(haug_hbm.at[selsrc.at[pl.ds(off, KB)]], hb, sh)
        pltpu.async_copy(auxd_hbm.at[seldst.at[pl.ds(off, KB)]], ab, sa)

    def g_wait(hb, ab, sh, sa):
        pltpu.make_async_copy(haug_hbm.at[selsrc.at[pl.ds(0, KB)]], hb,
                              sh).wait()
        pltpu.make_async_copy(auxd_hbm.at[seldst.at[pl.ds(0, KB)]], ab,
                              sa).wait()

    def process(b, hb, ab):
        off = jnp.minimum(b * KB, CAP - KB)

        def edge_body(j):
            jv = jnp.full((16,), j, jnp.int32)
            dsel = plsc.load_gather(seldst, [jv + off])
            ldv = jnp.clip(dsel - base, 0, ROWS - 1)
            caux = hb[j, pl.ds(256, 32)]
            e_aux, _ = plsc.unpack(caux, format=plsc.PackFormat.INTERLEAVED)
            z = e_aux + ab[j]
            z = jnp.where(z > 0, z, 0.2 * z)
            w = jnp.exp(z)
            valid = ((iota16 >= 4) & (iota16 < 8)
                     & (jnp.full((16,), off + j, jnp.int32) < count))
            w = jnp.where(valid, w, 0.0)
            wbc = [_rgather(w, jnp.full((16,), 4 + h, jnp.int32))
                   for h in range(4)]
            wsh = _rgather(w, jnp.minimum(iota16 + 4, 15))
            for c in range(8):
                hv = hb[j, pl.ds(c * 32, 32)]
                ev, ov = plsc.unpack(hv, format=plsc.PackFormat.INTERLEAVED)
                wj = wbc[c // 2]
                plsc.addupdate_scatter(acc, [ldv, c * 32 + iota16], ev * wj)
                plsc.addupdate_scatter(acc, [ldv, c * 32 + 16 + iota16],
                                       ov * wj)
            plsc.addupdate_scatter(acc, [ldv, 256 + iota16], e_aux * wsh)

        plsc.parallel_loop(0, KB, unroll=8)(edge_body)

    nb = (count + (KB - 1)) // KB
    npair = (nb + 1) // 2

    g_issue(0, h0, a0, semh0, sema0)

    def pair_body(i, _):
        b0 = 2 * i
        g_issue(b0 + 1, h1, a1, semh1, sema1)
        g_wait(h0, a0, semh0, sema0)
        process(b0, h0, a0)
        g_issue(b0 + 2, h0, a0, semh0, sema0)
        g_wait(h1, a1, semh1, sema1)
        process(b0 + 1, h1, a1)
        return 0

    lax.fori_loop(0, npair, pair_body, 0)
    g_wait(h0, a0, semh0, sema0)

    pltpu.sync_copy(acc, out_hbm.at[pl.ds(base, ROWS)])


def _phase2(edge_index, haug, auxd):
    zi = jnp.zeros((CAP,), jnp.int32)
    zf = jnp.zeros((ROWS, AW), jnp.float32)
    mesh = plsc.VectorSubcoreMesh(core_axis_name="c", subcore_axis_name="s")
    fn = pl.kernel(
        _sc_body,
        out_type=jax.ShapeDtypeStruct((NPAD, AW), jnp.float32),
        mesh=mesh,
        compiler_params=pltpu.CompilerParams(needs_layout_passes=False,
                                             use_tc_tiling_on_sc=False),
        scratch_types=[
            pltpu.VMEM((CAP,), jnp.int32),
            pltpu.VMEM((CAP,), jnp.int32),
            pltpu.VMEM((EB,), jnp.int32),
            pltpu.VMEM((EB,), jnp.int32),
            pltpu.VMEM((KB, AW), jnp.bfloat16),
            pltpu.VMEM((KB, AW), jnp.bfloat16),
            pltpu.VMEM((KB, 16), jnp.float32),
            pltpu.VMEM((KB, 16), jnp.float32),
            pltpu.VMEM((KB, 16), jnp.float32),
            pltpu.VMEM((ROWS, AW), jnp.float32),
            pltpu.SemaphoreType.DMA,
            pltpu.SemaphoreType.DMA,
            pltpu.SemaphoreType.DMA,
            pltpu.SemaphoreType.DMA,
            pltpu.SemaphoreType.DMA,
            pltpu.SemaphoreType.DMA,
        ],
    )
    return fn(edge_index, haug, auxd, zi, zf)


def _p3_body(acc_ref, aux2_ref, haug_ref, q_ref, s_ref, rb_ref, d_ref,
             out_ref):
    accb = acc_ref[...]
    num = jnp.dot(accb[:, :256], d_ref[...],
                  preferred_element_type=jnp.float32)
    den = accb[:, 256:260]
    S = s_ref[...]
    rinv = 1.0 / (den + 1e-16)
    agg = num * jnp.dot(rinv, S, preferred_element_type=jnp.float32)
    sr0 = jnp.dot(agg, q_ref[...], preferred_element_type=jnp.float32)[:, :4]
    aux2 = aux2_ref[...]
    rb = rb_ref[...]
    z0 = aux2[:, 0:4] + sr0
    z1 = aux2[:, 4:8] + aux2[:, 8:12]
    l0 = jnp.where(z0 > 0, z0, 0.2 * z0) + rb[:, 0:4]
    l1 = jnp.where(z1 > 0, z1, 0.2 * z1) + rb[:, 4:8]
    b0 = jax.nn.sigmoid(l0 - l1)
    b1 = 1.0 - b0
    h = haug_ref[...]
    out = agg * jnp.dot(b0, S, preferred_element_type=jnp.float32) + \
        h * jnp.dot(b1, S, preferred_element_type=jnp.float32)
    out_ref[...] = jnp.maximum(out, 0.0)


def _phase3(acc, aux2, hf, Q, S, rbv, Dsel):
    grid = (N // BLK,)
    return pl.pallas_call(
        _p3_body,
        grid=grid,
        in_specs=[
            pl.BlockSpec((BLK, AW), lambda i: (i, 0)),
            pl.BlockSpec((BLK, 16), lambda i: (i, 0)),
            pl.BlockSpec((BLK, D_OUT), lambda i: (i, 0)),
            pl.BlockSpec((D_OUT, 8), lambda i: (0, 0)),
            pl.BlockSpec((4, D_OUT), lambda i: (0, 0)),
            pl.BlockSpec((1, 8), lambda i: (0, 0)),
            pl.BlockSpec((D_OUT, D_OUT), lambda i: (0, 0)),
        ],
        out_specs=pl.BlockSpec((BLK, D_OUT), lambda i: (i, 0)),
        out_shape=jax.ShapeDtypeStruct((N, D_OUT), jnp.float32),
    )(acc, aux2, hf, Q, S, rbv, Dsel)


def _blockdiag(v):
    eye = jnp.eye(HEADS, dtype=jnp.float32)
    return (v[:, :, None] * eye[:, None, :]).reshape(HEADS * OUT_CH, HEADS)


_S_SEL = np.kron(np.eye(4, dtype=np.float32), np.ones((1, 64), np.float32))
_ILV = np.zeros((8, 32), np.float32)
for _k in range(8):
    _ILV[_k, 2 * _k] = 1.0
_PERM = np.zeros((256, 256), np.float32)
for _c in range(8):
    for _k in range(16):
        _PERM[32 * _c + _k, 32 * _c + 2 * _k] = 1.0
        _PERM[32 * _c + 16 + _k, 32 * _c + 2 * _k + 1] = 1.0


def kernel(x, edge_index, global_node_index, W, b, attn, rel_attn_l,
           rel_attn_r, rel_bias):
    b2d = b.reshape(1, D_OUT)
    Pmat = jnp.concatenate([
        _blockdiag(attn[:, :OUT_CH]),
        _blockdiag(attn[:, OUT_CH:]),
        _blockdiag(rel_attn_l[0]),
        _blockdiag(rel_attn_l[1]),
        _blockdiag(rel_attn_r[1]),
        jnp.zeros((D_IN, 12), jnp.float32),
    ], axis=1)
    Q = jnp.concatenate([_blockdiag(rel_attn_r[0]),
                         jnp.zeros((D_IN, 4), jnp.float32)], axis=1)
    rbv = jnp.concatenate([jnp.full((4,), rel_bias[0], jnp.float32),
                           jnp.full((4,), rel_bias[1], jnp.float32)]
                          ).reshape(1, 8)

    hbf, hf, auxd, aux2 = _phase1(x, W, b2d, Pmat, jnp.asarray(_ILV))
    accp = _phase2(edge_index, hbf, auxd)
    out = _phase3(accp, aux2, hf, Q, jnp.asarray(_S_SEL), rbv,
                  jnp.asarray(_PERM))
    return out


# speedup vs baseline: 1.3659x; 1.3659x over previous
import functools

import jax
import jax.numpy as jnp
import numpy as np
from jax import lax
from jax.experimental import pallas as pl
from jax.experimental.pallas import tpu as pltpu
from jax.experimental.pallas import tpu_sc as plsc

N = 10000
E = 160000
D_IN = 256
D_OUT = 256
HEADS = 4
OUT_CH = 64

NW = 32
ROWS = 320
NPAD = NW * ROWS
AW = 288
CAP = 5760
KB = 32
EB = 4000

BLK = 400


def _p1_body(x_ref, w_ref, b_ref, p_ref, ilv_ref, hbf_ref, hf_ref, auxd_ref,
             aux2_ref):
    xb = x_ref[...]
    h = jnp.dot(xb, w_ref[...], preferred_element_type=jnp.float32)
    h = h + b_ref[...]
    aux = jnp.dot(h, p_ref[...], preferred_element_type=jnp.float32)
    ones = jnp.ones((BLK, 4), jnp.float32)
    z4 = jnp.zeros((BLK, 4), jnp.float32)
    z8 = jnp.zeros((BLK, 8), jnp.float32)
    vale = jnp.concatenate([ones, aux[:, 0:4]], axis=1)
    inter = jnp.dot(vale, ilv_ref[...], preferred_element_type=jnp.float32)
    hbf_ref[...] = jnp.concatenate([h, inter], axis=1).astype(jnp.bfloat16)
    hf_ref[...] = h
    auxd_ref[...] = jnp.concatenate([z4, aux[:, 4:8], z8], axis=1)
    aux2_ref[...] = aux[:, 8:24]


def _phase1(x, W, b2d, Pmat, Ilv):
    grid = (N // BLK,)
    return pl.pallas_call(
        _p1_body,
        grid=grid,
        in_specs=[
            pl.BlockSpec((BLK, D_IN), lambda i: (i, 0)),
            pl.BlockSpec((D_IN, D_OUT), lambda i: (0, 0)),
            pl.BlockSpec((1, D_OUT), lambda i: (0, 0)),
            pl.BlockSpec((D_IN, 32), lambda i: (0, 0)),
            pl.BlockSpec((8, 32), lambda i: (0, 0)),
        ],
        out_specs=[
            pl.BlockSpec((BLK, AW), lambda i: (i, 0)),
            pl.BlockSpec((BLK, D_OUT), lambda i: (i, 0)),
            pl.BlockSpec((BLK, 16), lambda i: (i, 0)),
            pl.BlockSpec((BLK, 16), lambda i: (i, 0)),
        ],
        out_shape=[
            jax.ShapeDtypeStruct((N, AW), jnp.bfloat16),
            jax.ShapeDtypeStruct((N, D_OUT), jnp.float32),
            jax.ShapeDtypeStruct((N, 16), jnp.float32),
            jax.ShapeDtypeStruct((N, 16), jnp.float32),
        ],
    )(x, W, b2d, Pmat, Ilv)


def _sc_body(edge_hbm, haug_hbm, auxd_hbm, zi_hbm, zf_hbm,
             out_hbm, selsrc, seldst, esrc, edst, h0, h1, a0, a1, wbuf,
             acc, semh0, semh1, sema0, sema1, semes, semed):
    cid = lax.axis_index("c")
    sid = lax.axis_index("s")
    wid = sid * 2 + cid
    base = wid * ROWS
    iota16 = lax.broadcasted_iota(jnp.int32, (16,), 0)

    pltpu.sync_copy(zf_hbm, acc)
    pltpu.sync_copy(zi_hbm, selsrc)
    pltpu.sync_copy(zi_hbm, seldst)

    def blk_body(bi, cnt):
        cps = pltpu.async_copy(edge_hbm.at[0, pl.ds(bi * EB, EB)], esrc,
                               semes)
        cpd = pltpu.async_copy(edge_hbm.at[1, pl.ds(bi * EB, EB)], edst,
                               semed)
        cps.wait()
        cpd.wait()

        def in_body(t, cntv):
            d = edst[pl.ds(t * 16, 16)]
            s = esrc[pl.ds(t * 16, 16)]
            ld = d - base
            m = (ld >= 0) & (ld < ROWS)
            pcv = plsc.all_reduce_population_count(m)
            pos = plsc.cumsum(m.astype(jnp.int32))
            idx = jnp.minimum(cntv, CAP - 16) + pos - 1
            plsc.store_scatter(selsrc, [idx], s, mask=m)
            plsc.store_scatter(seldst, [idx], d, mask=m)
            return cntv + pcv

        return plsc.parallel_loop(0, EB // 16, unroll=4, carry=cnt)(in_body)

    cnt0 = jnp.zeros((16,), jnp.int32)
    countv = lax.fori_loop(0, E // EB, blk_body, cnt0)
    count = jnp.minimum(countv[0], CAP)

    def g_issue(b, hb, ab, sh, sa):
        off = jnp.minimum(b * KB, CAP - KB)
        pltpu.async_copy(haug_hbm.at[selsrc.at[pl.ds(off, KB)]], hb, sh)
        pltpu.async_copy(auxd_hbm.at[seldst.at[pl.ds(off, KB)]], ab, sa)

    def g_wait(hb, ab, sh, sa):
        pltpu.make_async_copy(haug_hbm.at[selsrc.at[pl.ds(0, KB)]], hb,
                              sh).wait()
        pltpu.make_async_copy(auxd_hbm.at[seldst.at[pl.ds(0, KB)]], ab,
                              sa).wait()

    def process(b, hb, ab):
        off = jnp.minimum(b * KB, CAP - KB)

        def edge_body(j):
            jv = jnp.full((16,), j, jnp.int32)
            dsel = plsc.load_gather(seldst, [jv + off])
            ldv = jnp.clip(dsel - base, 0, ROWS - 1)
            caux = hb[j, pl.ds(256, 32)]
            e_aux, _ = plsc.unpack(caux, format=plsc.PackFormat.INTERLEAVED)
            z = e_aux + ab[j]
            z = jnp.where(z > 0, z, 0.2 * z)
            w = jnp.exp(z)
            valid = ((iota16 >= 4) & (iota16 < 8)
                     & (jnp.full((16,), off + j, jnp.int32) < count))
            w = jnp.where(valid, w, 0.0)
            wbuf[j] = w
            wbc = [plsc.load_gather(wbuf,
                                    [jv, jnp.full((16,), 4 + h, jnp.int32)])
                   for h in range(4)]
            wsh = plsc.load_gather(wbuf, [jv, jnp.minimum(iota16 + 4, 15)])
            for c in range(8):
                hv = hb[j, pl.ds(c * 32, 32)]
                ev, ov = plsc.unpack(hv, format=plsc.PackFormat.INTERLEAVED)
                wj = wbc[c // 2]
                plsc.addupdate_scatter(acc, [ldv, c * 32 + iota16], ev * wj)
                plsc.addupdate_scatter(acc, [ldv, c * 32 + 16 + iota16],
                                       ov * wj)
            plsc.addupdate_scatter(acc, [ldv, 256 + iota16], e_aux * wsh)

        plsc.parallel_loop(0, KB, unroll=8)(edge_body)

    nb = (count + (KB - 1)) // KB
    npair = (nb + 1) // 2

    g_issue(0, h0, a0, semh0, sema0)

    def pair_body(i, _):
        b0 = 2 * i
        g_issue(b0 + 1, h1, a1, semh1, sema1)
        g_wait(h0, a0, semh0, sema0)
        process(b0, h0, a0)
        g_issue(b0 + 2, h0, a0, semh0, sema0)
        g_wait(h1, a1, semh1, sema1)
        process(b0 + 1, h1, a1)
        return 0

    lax.fori_loop(0, npair, pair_body, 0)
    g_wait(h0, a0, semh0, sema0)

    pltpu.sync_copy(acc, out_hbm.at[pl.ds(base, ROWS)])


def _phase2(edge_index, haug, auxd):
    zi = jnp.zeros((CAP,), jnp.int32)
    zf = jnp.zeros((ROWS, AW), jnp.float32)
    mesh = plsc.VectorSubcoreMesh(core_axis_name="c", subcore_axis_name="s")
    fn = pl.kernel(
        _sc_body,
        out_type=jax.ShapeDtypeStruct((NPAD, AW), jnp.float32),
        mesh=mesh,
        compiler_params=pltpu.CompilerParams(needs_layout_passes=False,
                                             use_tc_tiling_on_sc=False),
        scratch_types=[
            pltpu.VMEM((CAP,), jnp.int32),
            pltpu.VMEM((CAP,), jnp.int32),
            pltpu.VMEM((EB,), jnp.int32),
            pltpu.VMEM((EB,), jnp.int32),
            pltpu.VMEM((KB, AW), jnp.bfloat16),
            pltpu.VMEM((KB, AW), jnp.bfloat16),
            pltpu.VMEM((KB, 16), jnp.float32),
            pltpu.VMEM((KB, 16), jnp.float32),
            pltpu.VMEM((KB, 16), jnp.float32),
            pltpu.VMEM((ROWS, AW), jnp.float32),
            pltpu.SemaphoreType.DMA,
            pltpu.SemaphoreType.DMA,
            pltpu.SemaphoreType.DMA,
            pltpu.SemaphoreType.DMA,
            pltpu.SemaphoreType.DMA,
            pltpu.SemaphoreType.DMA,
        ],
    )
    return fn(edge_index, haug, auxd, zi, zf)


def _p3_body(acc_ref, aux2_ref, haug_ref, q_ref, s_ref, rb_ref, d_ref,
             out_ref):
    accb = acc_ref[...]
    num = jnp.dot(accb[:, :256], d_ref[...],
                  preferred_element_type=jnp.float32)
    den = accb[:, 256:260]
    S = s_ref[...]
    rinv = 1.0 / (den + 1e-16)
    agg = num * jnp.dot(rinv, S, preferred_element_type=jnp.float32)
    sr0 = jnp.dot(agg, q_ref[...], preferred_element_type=jnp.float32)[:, :4]
    aux2 = aux2_ref[...]
    rb = rb_ref[...]
    z0 = aux2[:, 0:4] + sr0
    z1 = aux2[:, 4:8] + aux2[:, 8:12]
    l0 = jnp.where(z0 > 0, z0, 0.2 * z0) + rb[:, 0:4]
    l1 = jnp.where(z1 > 0, z1, 0.2 * z1) + rb[:, 4:8]
    b0 = jax.nn.sigmoid(l0 - l1)
    b1 = 1.0 - b0
    h = haug_ref[...]
    out = agg * jnp.dot(b0, S, preferred_element_type=jnp.float32) + \
        h * jnp.dot(b1, S, preferred_element_type=jnp.float32)
    out_ref[...] = jnp.maximum(out, 0.0)


def _phase3(acc, aux2, hf, Q, S, rbv, Dsel):
    grid = (N // BLK,)
    return pl.pallas_call(
        _p3_body,
        grid=grid,
        in_specs=[
            pl.BlockSpec((BLK, AW), lambda i: (i, 0)),
            pl.BlockSpec((BLK, 16), lambda i: (i, 0)),
            pl.BlockSpec((BLK, D_OUT), lambda i: (i, 0)),
            pl.BlockSpec((D_OUT, 8), lambda i: (0, 0)),
            pl.BlockSpec((4, D_OUT), lambda i: (0, 0)),
            pl.BlockSpec((1, 8), lambda i: (0, 0)),
            pl.BlockSpec((D_OUT, D_OUT), lambda i: (0, 0)),
        ],
        out_specs=pl.BlockSpec((BLK, D_OUT), lambda i: (i, 0)),
        out_shape=jax.ShapeDtypeStruct((N, D_OUT), jnp.float32),
    )(acc, aux2, hf, Q, S, rbv, Dsel)


def _blockdiag(v):
    eye = jnp.eye(HEADS, dtype=jnp.float32)
    return (v[:, :, None] * eye[:, None, :]).reshape(HEADS * OUT_CH, HEADS)


_S_SEL = np.kron(np.eye(4, dtype=np.float32), np.ones((1, 64), np.float32))
_ILV = np.zeros((8, 32), np.float32)
for _k in range(8):
    _ILV[_k, 2 * _k] = 1.0
_PERM = np.zeros((256, 256), np.float32)
for _c in range(8):
    for _k in range(16):
        _PERM[32 * _c + _k, 32 * _c + 2 * _k] = 1.0
        _PERM[32 * _c + 16 + _k, 32 * _c + 2 * _k + 1] = 1.0


def kernel(x, edge_index, global_node_index, W, b, attn, rel_attn_l,
           rel_attn_r, rel_bias):
    b2d = b.reshape(1, D_OUT)
    Pmat = jnp.concatenate([
        _blockdiag(attn[:, :OUT_CH]),
        _blockdiag(attn[:, OUT_CH:]),
        _blockdiag(rel_attn_l[0]),
        _blockdiag(rel_attn_l[1]),
        _blockdiag(rel_attn_r[1]),
        jnp.zeros((D_IN, 12), jnp.float32),
    ], axis=1)
    Q = jnp.concatenate([_blockdiag(rel_attn_r[0]),
                         jnp.zeros((D_IN, 4), jnp.float32)], axis=1)
    rbv = jnp.concatenate([jnp.full((4,), rel_bias[0], jnp.float32),
                           jnp.full((4,), rel_bias[1], jnp.float32)]
                          ).reshape(1, 8)

    hbf, hf, auxd, aux2 = _phase1(x, W, b2d, Pmat, jnp.asarray(_ILV))
    accp = _phase2(edge_index, hbf, auxd)
    out = _phase3(accp, aux2, hf, Q, jnp.asarray(_S_SEL), rbv,
                  jnp.asarray(_PERM))
    return out


# speedup vs baseline: 1.4111x; 1.0331x over previous
import functools

import jax
import jax.numpy as jnp
import numpy as np
from jax import lax
from jax.experimental import pallas as pl
from jax.experimental.pallas import tpu as pltpu
from jax.experimental.pallas import tpu_sc as plsc

N = 10000
E = 160000
D_IN = 256
D_OUT = 256
HEADS = 4
OUT_CH = 64

NW = 32
ROWS = 320
NPAD = NW * ROWS
AW = 288
CAP = 5760
KB = 32
EB = 2000

BLK = 400


def _p1_body(x_ref, w_ref, b_ref, p_ref, ilv_ref, hbf_ref, hf_ref, auxd_ref,
             aux2_ref):
    xb = x_ref[...]
    h = jnp.dot(xb, w_ref[...], preferred_element_type=jnp.float32)
    h = h + b_ref[...]
    aux = jnp.dot(h, p_ref[...], preferred_element_type=jnp.float32)
    ones = jnp.ones((BLK, 4), jnp.float32)
    z4 = jnp.zeros((BLK, 4), jnp.float32)
    z8 = jnp.zeros((BLK, 8), jnp.float32)
    vale = jnp.concatenate([ones, aux[:, 0:4]], axis=1)
    inter = jnp.dot(vale, ilv_ref[...], preferred_element_type=jnp.float32)
    hbf_ref[...] = jnp.concatenate([h, inter], axis=1).astype(jnp.bfloat16)
    hf_ref[...] = h
    auxd_ref[...] = jnp.concatenate([z4, aux[:, 4:8], z8], axis=1)
    aux2_ref[...] = aux[:, 8:24]


def _phase1(x, W, b2d, Pmat, Ilv):
    grid = (N // BLK,)
    return pl.pallas_call(
        _p1_body,
        grid=grid,
        in_specs=[
            pl.BlockSpec((BLK, D_IN), lambda i: (i, 0)),
            pl.BlockSpec((D_IN, D_OUT), lambda i: (0, 0)),
            pl.BlockSpec((1, D_OUT), lambda i: (0, 0)),
            pl.BlockSpec((D_IN, 32), lambda i: (0, 0)),
            pl.BlockSpec((8, 32), lambda i: (0, 0)),
        ],
        out_specs=[
            pl.BlockSpec((BLK, AW), lambda i: (i, 0)),
            pl.BlockSpec((BLK, D_OUT), lambda i: (i, 0)),
            pl.BlockSpec((BLK, 16), lambda i: (i, 0)),
            pl.BlockSpec((BLK, 16), lambda i: (i, 0)),
        ],
        out_shape=[
            jax.ShapeDtypeStruct((N, AW), jnp.bfloat16),
            jax.ShapeDtypeStruct((N, D_OUT), jnp.float32),
            jax.ShapeDtypeStruct((N, 16), jnp.float32),
            jax.ShapeDtypeStruct((N, 16), jnp.float32),
        ],
    )(x, W, b2d, Pmat, Ilv)


def _sc_body(edge_hbm, haug_hbm, auxd_hbm, zi_hbm, zf_hbm,
             out_hbm, selsrc, seldst, esrc, edst, esrc1, edst1, h0, h1, a0,
             a1, wbuf, acc, semh0, semh1, sema0, sema1, semes, semed, semes1,
             semed1):
    cid = lax.axis_index("c")
    sid = lax.axis_index("s")
    wid = sid * 2 + cid
    base = wid * ROWS
    iota16 = lax.broadcasted_iota(jnp.int32, (16,), 0)

    pltpu.sync_copy(zf_hbm, acc)
    pltpu.sync_copy(zi_hbm, selsrc)
    pltpu.sync_copy(zi_hbm, seldst)

    def f_issue(b, eb_s, eb_d, ss, sd):
        off = jnp.minimum(b * EB, E - EB)
        pltpu.async_copy(edge_hbm.at[0, pl.ds(off, EB)], eb_s, ss)
        pltpu.async_copy(edge_hbm.at[1, pl.ds(off, EB)], eb_d, sd)

    def f_wait(eb_s, eb_d, ss, sd):
        pltpu.make_async_copy(edge_hbm.at[0, pl.ds(0, EB)], eb_s, ss).wait()
        pltpu.make_async_copy(edge_hbm.at[1, pl.ds(0, EB)], eb_d, sd).wait()

    def f_scan(eb_s, eb_d, cnt):
        def in_body(t, cntv):
            d = eb_d[pl.ds(t * 16, 16)]
            s = eb_s[pl.ds(t * 16, 16)]
            ld = d - base
            m = (ld >= 0) & (ld < ROWS)
            pcv = plsc.all_reduce_population_count(m)
            pos = plsc.cumsum(m.astype(jnp.int32))
            idx = jnp.minimum(cntv, CAP - 16) + pos - 1
            plsc.store_scatter(selsrc, [idx], s, mask=m)
            plsc.store_scatter(seldst, [idx], d, mask=m)
            return cntv + pcv

        return plsc.parallel_loop(0, EB // 16, unroll=4, carry=cnt)(in_body)

    cnt0 = jnp.zeros((16,), jnp.int32)
    f_issue(0, esrc, edst, semes, semed)

    def fpair(i, cnt):
        b0 = 2 * i
        f_issue(b0 + 1, esrc1, edst1, semes1, semed1)
        f_wait(esrc, edst, semes, semed)
        cnt = f_scan(esrc, edst, cnt)
        f_issue(b0 + 2, esrc, edst, semes, semed)
        f_wait(esrc1, edst1, semes1, semed1)
        cnt = f_scan(esrc1, edst1, cnt)
        return cnt

    countv = lax.fori_loop(0, E // EB // 2, fpair, cnt0)
    f_wait(esrc, edst, semes, semed)
    count = jnp.minimum(countv[0], CAP)

    def g_issue(b, hb, ab, sh, sa):
        off = jnp.minimum(b * KB, CAP - KB)
        pltpu.async_copy(haug_hbm.at[selsrc.at[pl.ds(off, KB)]], hb, sh)
        pltpu.async_copy(auxd_hbm.at[seldst.at[pl.ds(off, KB)]], ab, sa)

    def g_wait(hb, ab, sh, sa):
        pltpu.make_async_copy(haug_hbm.at[selsrc.at[pl.ds(0, KB)]], hb,
                              sh).wait()
        pltpu.make_async_copy(auxd_hbm.at[seldst.at[pl.ds(0, KB)]], ab,
                              sa).wait()

    def process(b, hb, ab):
        off = jnp.minimum(b * KB, CAP - KB)

        def edge_body(j):
            jv = jnp.full((16,), j, jnp.int32)
            dsel = plsc.load_gather(seldst, [jv + off])
            ldv = jnp.clip(dsel - base, 0, ROWS - 1)
            caux = hb[j, pl.ds(256, 32)]
            e_aux, _ = plsc.unpack(caux, format=plsc.PackFormat.INTERLEAVED)
            z = e_aux + ab[j]
            z = jnp.where(z > 0, z, 0.2 * z)
            w = jnp.exp(z)
            valid = ((iota16 >= 4) & (iota16 < 8)
                     & (jnp.full((16,), off + j, jnp.int32) < count))
            w = jnp.where(valid, w, 0.0)
            wbuf[j] = w
            wbc = [plsc.load_gather(wbuf,
                                    [jv, jnp.full((16,), 4 + h, jnp.int32)])
                   for h in range(4)]
            wsh = plsc.load_gather(wbuf, [jv, jnp.minimum(iota16 + 4, 15)])
            for c in range(8):
                hv = hb[j, pl.ds(c * 32, 32)]
                ev, ov = plsc.unpack(hv, format=plsc.PackFormat.INTERLEAVED)
                wj = wbc[c // 2]
                plsc.addupdate_scatter(acc, [ldv, c * 32 + iota16], ev * wj)
                plsc.addupdate_scatter(acc, [ldv, c * 32 + 16 + iota16],
                                       ov * wj)
            plsc.addupdate_scatter(acc, [ldv, 256 + iota16], e_aux * wsh)

        plsc.parallel_loop(0, KB, unroll=8)(edge_body)

    nb = (count + (KB - 1)) // KB
    npair = (nb + 1) // 2

    g_issue(0, h0, a0, semh0, sema0)

    def pair_body(i, _):
        b0 = 2 * i
        g_issue(b0 + 1, h1, a1, semh1, sema1)
        g_wait(h0, a0, semh0, sema0)
        process(b0, h0, a0)
        g_issue(b0 + 2, h0, a0, semh0, sema0)
        g_wait(h1, a1, semh1, sema1)
        process(b0 + 1, h1, a1)
        return 0

    lax.fori_loop(0, npair, pair_body, 0)
    g_wait(h0, a0, semh0, sema0)

    pltpu.sync_copy(acc, out_hbm.at[pl.ds(base, ROWS)])


def _phase2(edge_index, haug, auxd):
    zi = jnp.zeros((CAP,), jnp.int32)
    zf = jnp.zeros((ROWS, AW), jnp.float32)
    mesh = plsc.VectorSubcoreMesh(core_axis_name="c", subcore_axis_name="s")
    fn = pl.kernel(
        _sc_body,
        out_type=jax.ShapeDtypeStruct((NPAD, AW), jnp.float32),
        mesh=mesh,
        compiler_params=pltpu.CompilerParams(needs_layout_passes=False,
                                             use_tc_tiling_on_sc=False),
        scratch_types=[
            pltpu.VMEM((CAP,), jnp.int32),
            pltpu.VMEM((CAP,), jnp.int32),
            pltpu.VMEM((EB,), jnp.int32),
            pltpu.VMEM((EB,), jnp.int32),
            pltpu.VMEM((EB,), jnp.int32),
            pltpu.VMEM((EB,), jnp.int32),
            pltpu.VMEM((KB, AW), jnp.bfloat16),
            pltpu.VMEM((KB, AW), jnp.bfloat16),
            pltpu.VMEM((KB, 16), jnp.float32),
            pltpu.VMEM((KB, 16), jnp.float32),
            pltpu.VMEM((KB, 16), jnp.float32),
            pltpu.VMEM((ROWS, AW), jnp.float32),
            pltpu.SemaphoreType.DMA,
            pltpu.SemaphoreType.DMA,
            pltpu.SemaphoreType.DMA,
            pltpu.SemaphoreType.DMA,
            pltpu.SemaphoreType.DMA,
            pltpu.SemaphoreType.DMA,
            pltpu.SemaphoreType.DMA,
            pltpu.SemaphoreType.DMA,
        ],
    )
    return fn(edge_index, haug, auxd, zi, zf)


def _p3_body(acc_ref, aux2_ref, haug_ref, q_ref, s_ref, rb_ref, d_ref,
             out_ref):
    accb = acc_ref[...]
    num = jnp.dot(accb[:, :256], d_ref[...],
                  preferred_element_type=jnp.float32)
    den = accb[:, 256:260]
    S = s_ref[...]
    rinv = 1.0 / (den + 1e-16)
    agg = num * jnp.dot(rinv, S, preferred_element_type=jnp.float32)
    sr0 = jnp.dot(agg, q_ref[...], preferred_element_type=jnp.float32)[:, :4]
    aux2 = aux2_ref[...]
    rb = rb_ref[...]
    z0 = aux2[:, 0:4] + sr0
    z1 = aux2[:, 4:8] + aux2[:, 8:12]
    l0 = jnp.where(z0 > 0, z0, 0.2 * z0) + rb[:, 0:4]
    l1 = jnp.where(z1 > 0, z1, 0.2 * z1) + rb[:, 4:8]
    b0 = jax.nn.sigmoid(l0 - l1)
    b1 = 1.0 - b0
    h = haug_ref[...]
    out = agg * jnp.dot(b0, S, preferred_element_type=jnp.float32) + \
        h * jnp.dot(b1, S, preferred_element_type=jnp.float32)
    out_ref[...] = jnp.maximum(out, 0.0)


def _phase3(acc, aux2, hf, Q, S, rbv, Dsel):
    grid = (N // BLK,)
    return pl.pallas_call(
        _p3_body,
        grid=grid,
        in_specs=[
            pl.BlockSpec((BLK, AW), lambda i: (i, 0)),
            pl.BlockSpec((BLK, 16), lambda i: (i, 0)),
            pl.BlockSpec((BLK, D_OUT), lambda i: (i, 0)),
            pl.BlockSpec((D_OUT, 8), lambda i: (0, 0)),
            pl.BlockSpec((4, D_OUT), lambda i: (0, 0)),
            pl.BlockSpec((1, 8), lambda i: (0, 0)),
            pl.BlockSpec((D_OUT, D_OUT), lambda i: (0, 0)),
        ],
        out_specs=pl.BlockSpec((BLK, D_OUT), lambda i: (i, 0)),
        out_shape=jax.ShapeDtypeStruct((N, D_OUT), jnp.float32),
    )(acc, aux2, hf, Q, S, rbv, Dsel)


def _blockdiag(v):
    eye = jnp.eye(HEADS, dtype=jnp.float32)
    return (v[:, :, None] * eye[:, None, :]).reshape(HEADS * OUT_CH, HEADS)


_S_SEL = np.kron(np.eye(4, dtype=np.float32), np.ones((1, 64), np.float32))
_ILV = np.zeros((8, 32), np.float32)
for _k in range(8):
    _ILV[_k, 2 * _k] = 1.0
_PERM = np.zeros((256, 256), np.float32)
for _c in range(8):
    for _k in range(16):
        _PERM[32 * _c + _k, 32 * _c + 2 * _k] = 1.0
        _PERM[32 * _c + 16 + _k, 32 * _c + 2 * _k + 1] = 1.0


def kernel(x, edge_index, global_node_index, W, b, attn, rel_attn_l,
           rel_attn_r, rel_bias):
    b2d = b.reshape(1, D_OUT)
    Pmat = jnp.concatenate([
        _blockdiag(attn[:, :OUT_CH]),
        _blockdiag(attn[:, OUT_CH:]),
        _blockdiag(rel_attn_l[0]),
        _blockdiag(rel_attn_l[1]),
        _blockdiag(rel_attn_r[1]),
        jnp.zeros((D_IN, 12), jnp.float32),
    ], axis=1)
    Q = jnp.concatenate([_blockdiag(rel_attn_r[0]),
                         jnp.zeros((D_IN, 4), jnp.float32)], axis=1)
    rbv = jnp.concatenate([jnp.full((4,), rel_bias[0], jnp.float32),
                           jnp.full((4,), rel_bias[1], jnp.float32)]
                          ).reshape(1, 8)

    hbf, hf, auxd, aux2 = _phase1(x, W, b2d, Pmat, jnp.asarray(_ILV))
    accp = _phase2(edge_index, hbf, auxd)
    out = _phase3(accp, aux2, hf, Q, jnp.asarray(_S_SEL), rbv,
                  jnp.asarray(_PERM))
    return out


# speedup vs baseline: 1.4127x; 1.0012x over previous
import jax
import jax.numpy as jnp
import numpy as np
from jax import lax
from jax.experimental import pallas as pl
from jax.experimental.pallas import tpu as pltpu
from jax.experimental.pallas import tpu_sc as plsc

N = 10000
E = 160000
D_IN = 256
D_OUT = 256
HEADS = 4
OUT_CH = 64

NW = 32
ROWS = 320
NPAD = NW * ROWS
AW = 288
CAP = 5760
KB = 32
EB = 2000

BLK = 400


def _p1_body(x_ref, w_ref, b_ref, p_ref, ilv_ref, hbf_ref, hf_ref, auxd_ref,
             aux2_ref):
    xb = x_ref[...]
    h = jnp.dot(xb, w_ref[...], preferred_element_type=jnp.float32)
    h = h + b_ref[...]
    aux = jnp.dot(h, p_ref[...], preferred_element_type=jnp.float32)
    ones = jnp.ones((BLK, 4), jnp.float32)
    z4 = jnp.zeros((BLK, 4), jnp.float32)
    z8 = jnp.zeros((BLK, 8), jnp.float32)
    vale = jnp.concatenate([ones, aux[:, 0:4]], axis=1)
    inter = jnp.dot(vale, ilv_ref[...], preferred_element_type=jnp.float32)
    hbf_ref[...] = jnp.concatenate([h, inter], axis=1).astype(jnp.bfloat16)
    hf_ref[...] = h
    auxd_ref[...] = jnp.concatenate([z4, aux[:, 4:8], z8], axis=1)
    aux2_ref[...] = aux[:, 8:24]


def _phase1(x, W, b2d, Pmat, Ilv):
    grid = (N // BLK,)
    return pl.pallas_call(
        _p1_body,
        grid=grid,
        in_specs=[
            pl.BlockSpec((BLK, D_IN), lambda i: (i, 0)),
            pl.BlockSpec((D_IN, D_OUT), lambda i: (0, 0)),
            pl.BlockSpec((1, D_OUT), lambda i: (0, 0)),
            pl.BlockSpec((D_IN, 32), lambda i: (0, 0)),
            pl.BlockSpec((8, 32), lambda i: (0, 0)),
        ],
        out_specs=[
            pl.BlockSpec((BLK, AW), lambda i: (i, 0)),
            pl.BlockSpec((BLK, D_OUT), lambda i: (i, 0)),
            pl.BlockSpec((BLK, 16), lambda i: (i, 0)),
            pl.BlockSpec((BLK, 16), lambda i: (i, 0)),
        ],
        out_shape=[
            jax.ShapeDtypeStruct((N, AW), jnp.bfloat16),
            jax.ShapeDtypeStruct((N, D_OUT), jnp.float32),
            jax.ShapeDtypeStruct((N, 16), jnp.float32),
            jax.ShapeDtypeStruct((N, 16), jnp.float32),
        ],
    )(x, W, b2d, Pmat, Ilv)


def _sc_body(edge_hbm, haug_hbm, auxd_hbm, zi_hbm, zf_hbm,
             out_hbm, selsrc, seldst, esrc, edst, esrc1, edst1, h0, h1, a0,
             a1, wbuf, acc, semh0, semh1, sema0, sema1, semes, semed, semes1,
             semed1):
    cid = lax.axis_index("c")
    sid = lax.axis_index("s")
    wid = sid * 2 + cid
    base = wid * ROWS
    iota16 = lax.broadcasted_iota(jnp.int32, (16,), 0)

    pltpu.sync_copy(zf_hbm, acc)
    pltpu.sync_copy(zi_hbm, selsrc)
    pltpu.sync_copy(zi_hbm, seldst)

    def f_issue(b, eb_s, eb_d, ss, sd):
        off = jnp.minimum(b * EB, E - EB)
        pltpu.async_copy(edge_hbm.at[0, pl.ds(off, EB)], eb_s, ss)
        pltpu.async_copy(edge_hbm.at[1, pl.ds(off, EB)], eb_d, sd)

    def f_wait(eb_s, eb_d, ss, sd):
        pltpu.make_async_copy(edge_hbm.at[0, pl.ds(0, EB)], eb_s, ss).wait()
        pltpu.make_async_copy(edge_hbm.at[1, pl.ds(0, EB)], eb_d, sd).wait()

    def f_scan(eb_s, eb_d, cnt):
        def in_body(t, cntv):
            d = eb_d[pl.ds(t * 16, 16)]
            s = eb_s[pl.ds(t * 16, 16)]
            ld = d - base
            m = (ld >= 0) & (ld < ROWS)
            pcv = plsc.all_reduce_population_count(m)
            pos = plsc.cumsum(m.astype(jnp.int32))
            idx = jnp.minimum(cntv, CAP - 16) + pos - 1
            plsc.store_scatter(selsrc, [idx], s, mask=m)
            plsc.store_scatter(seldst, [idx], d, mask=m)
            return cntv + pcv

        return plsc.parallel_loop(0, EB // 16, unroll=4, carry=cnt)(in_body)

    cnt0 = jnp.zeros((16,), jnp.int32)
    f_issue(0, esrc, edst, semes, semed)

    def fpair(i, cnt):
        b0 = 2 * i
        f_issue(b0 + 1, esrc1, edst1, semes1, semed1)
        f_wait(esrc, edst, semes, semed)
        cnt = f_scan(esrc, edst, cnt)
        f_issue(b0 + 2, esrc, edst, semes, semed)
        f_wait(esrc1, edst1, semes1, semed1)
        cnt = f_scan(esrc1, edst1, cnt)
        return cnt

    countv = lax.fori_loop(0, E // EB // 2, fpair, cnt0)
    f_wait(esrc, edst, semes, semed)
    count = jnp.minimum(countv[0], CAP)

    def g_issue(b, hb, ab, sh, sa):
        off = jnp.minimum(b * KB, CAP - KB)
        pltpu.async_copy(haug_hbm.at[selsrc.at[pl.ds(off, KB)]], hb, sh)
        pltpu.async_copy(auxd_hbm.at[seldst.at[pl.ds(off, KB)]], ab, sa)

    def g_wait(hb, ab, sh, sa):
        pltpu.make_async_copy(haug_hbm.at[selsrc.at[pl.ds(0, KB)]], hb,
                              sh).wait()
        pltpu.make_async_copy(auxd_hbm.at[seldst.at[pl.ds(0, KB)]], ab,
                              sa).wait()

    def process(b, hb, ab):
        off = jnp.minimum(b * KB, CAP - KB)

        def edge_body(j):
            jv = jnp.full((16,), j, jnp.int32)
            dsel = plsc.load_gather(seldst, [jv + off])
            ldv = jnp.clip(dsel - base, 0, ROWS - 1)
            caux = hb[j, pl.ds(256, 32)]
            e_aux, _ = plsc.unpack(caux, format=plsc.PackFormat.INTERLEAVED)
            z = e_aux + ab[j]
            z = jnp.where(z > 0, z, 0.2 * z)
            w = jnp.exp(z)
            valid = ((iota16 >= 4) & (iota16 < 8)
                     & (jnp.full((16,), off + j, jnp.int32) < count))
            w = jnp.where(valid, w, 0.0)
            wbuf[j] = w
            wbc = [plsc.load_gather(wbuf,
                                    [jv, jnp.full((16,), 4 + h, jnp.int32)])
                   for h in range(4)]
            wsh = plsc.load_gather(wbuf, [jv, jnp.minimum(iota16 + 4, 15)])
            for c in range(8):
                hv = hb[j, pl.ds(c * 32, 32)]
                ev, ov = plsc.unpack(hv, format=plsc.PackFormat.INTERLEAVED)
                wj = wbc[c // 2]
                plsc.addupdate_scatter(acc, [ldv, c * 32 + iota16], ev * wj)
                plsc.addupdate_scatter(acc, [ldv, c * 32 + 16 + iota16],
                                       ov * wj)
            plsc.addupdate_scatter(acc, [ldv, 256 + iota16], e_aux * wsh)

        plsc.parallel_loop(0, KB, unroll=8)(edge_body)

    nb = (count + (KB - 1)) // KB
    npair = (nb + 1) // 2

    g_issue(0, h0, a0, semh0, sema0)

    def pair_body(i, _):
        b0 = 2 * i
        g_issue(b0 + 1, h1, a1, semh1, sema1)
        g_wait(h0, a0, semh0, sema0)
        process(b0, h0, a0)
        g_issue(b0 + 2, h0, a0, semh0, sema0)
        g_wait(h1, a1, semh1, sema1)
        process(b0 + 1, h1, a1)
        return 0

    lax.fori_loop(0, npair, pair_body, 0)
    g_wait(h0, a0, semh0, sema0)

    pltpu.sync_copy(acc, out_hbm.at[pl.ds(base, ROWS)])


def _phase2(edge_index, haug, auxd):
    zi = jnp.zeros((CAP,), jnp.int32)
    zf = jnp.zeros((ROWS, AW), jnp.float32)
    mesh = plsc.VectorSubcoreMesh(core_axis_name="c", subcore_axis_name="s")
    fn = pl.kernel(
        _sc_body,
        out_type=jax.ShapeDtypeStruct((NPAD, AW), jnp.float32),
        mesh=mesh,
        compiler_params=pltpu.CompilerParams(needs_layout_passes=False,
                                             use_tc_tiling_on_sc=False),
        scratch_types=[
            pltpu.VMEM((CAP,), jnp.int32),
            pltpu.VMEM((CAP,), jnp.int32),
            pltpu.VMEM((EB,), jnp.int32),
            pltpu.VMEM((EB,), jnp.int32),
            pltpu.VMEM((EB,), jnp.int32),
            pltpu.VMEM((EB,), jnp.int32),
            pltpu.VMEM((KB, AW), jnp.bfloat16),
            pltpu.VMEM((KB, AW), jnp.bfloat16),
            pltpu.VMEM((KB, 16), jnp.float32),
            pltpu.VMEM((KB, 16), jnp.float32),
            pltpu.VMEM((KB, 16), jnp.float32),
            pltpu.VMEM((ROWS, AW), jnp.float32),
            pltpu.SemaphoreType.DMA,
            pltpu.SemaphoreType.DMA,
            pltpu.SemaphoreType.DMA,
            pltpu.SemaphoreType.DMA,
            pltpu.SemaphoreType.DMA,
            pltpu.SemaphoreType.DMA,
            pltpu.SemaphoreType.DMA,
            pltpu.SemaphoreType.DMA,
        ],
    )
    return fn(edge_index, haug, auxd, zi, zf)


def _p3_body(acc_ref, aux2_ref, haug_ref, q_ref, s_ref, rb_ref, d_ref,
             out_ref):
    accb = acc_ref[...]
    num = jnp.dot(accb[:, :256], d_ref[...],
                  preferred_element_type=jnp.float32)
    den = accb[:, 256:260]
    S = s_ref[...]
    rinv = 1.0 / (den + 1e-16)
    agg = num * jnp.dot(rinv, S, preferred_element_type=jnp.float32)
    sr0 = jnp.dot(agg, q_ref[...], preferred_element_type=jnp.float32)[:, :4]
    aux2 = aux2_ref[...]
    rb = rb_ref[...]
    z0 = aux2[:, 0:4] + sr0
    z1 = aux2[:, 4:8] + aux2[:, 8:12]
    l0 = jnp.where(z0 > 0, z0, 0.2 * z0) + rb[:, 0:4]
    l1 = jnp.where(z1 > 0, z1, 0.2 * z1) + rb[:, 4:8]
    b0 = jax.nn.sigmoid(l0 - l1)
    b1 = 1.0 - b0
    h = haug_ref[...]
    out = agg * jnp.dot(b0, S, preferred_element_type=jnp.float32) + \
        h * jnp.dot(b1, S, preferred_element_type=jnp.float32)
    out_ref[...] = jnp.maximum(out, 0.0)


def _phase3(acc, aux2, hf, Q, S, rbv, Dsel):
    grid = (N // BLK,)
    return pl.pallas_call(
        _p3_body,
        grid=grid,
        in_specs=[
            pl.BlockSpec((BLK, AW), lambda i: (i, 0)),
            pl.BlockSpec((BLK, 16), lambda i: (i, 0)),
            pl.BlockSpec((BLK, D_OUT), lambda i: (i, 0)),
            pl.BlockSpec((D_OUT, 8), lambda i: (0, 0)),
            pl.BlockSpec((4, D_OUT), lambda i: (0, 0)),
            pl.BlockSpec((1, 8), lambda i: (0, 0)),
            pl.BlockSpec((D_OUT, D_OUT), lambda i: (0, 0)),
        ],
        out_specs=pl.BlockSpec((BLK, D_OUT), lambda i: (i, 0)),
        out_shape=jax.ShapeDtypeStruct((N, D_OUT), jnp.float32),
    )(acc, aux2, hf, Q, S, rbv, Dsel)


def _blockdiag(v):
    eye = jnp.eye(HEADS, dtype=jnp.float32)
    return (v[:, :, None] * eye[:, None, :]).reshape(HEADS * OUT_CH, HEADS)


_S_SEL = np.kron(np.eye(4, dtype=np.float32), np.ones((1, 64), np.float32))
_ILV = np.zeros((8, 32), np.float32)
for _k in range(8):
    _ILV[_k, 2 * _k] = 1.0
_PERM = np.zeros((256, 256), np.float32)
for _c in range(8):
    for _k in range(16):
        _PERM[32 * _c + _k, 32 * _c + 2 * _k] = 1.0
        _PERM[32 * _c + 16 + _k, 32 * _c + 2 * _k + 1] = 1.0


def kernel(x, edge_index, global_node_index, W, b, attn, rel_attn_l,
           rel_attn_r, rel_bias):
    b2d = b.reshape(1, D_OUT)
    Pmat = jnp.concatenate([
        _blockdiag(attn[:, :OUT_CH]),
        _blockdiag(attn[:, OUT_CH:]),
        _blockdiag(rel_attn_l[0]),
        _blockdiag(rel_attn_l[1]),
        _blockdiag(rel_attn_r[1]),
        jnp.zeros((D_IN, 12), jnp.float32),
    ], axis=1)
    Q = jnp.concatenate([_blockdiag(rel_attn_r[0]),
                         jnp.zeros((D_IN, 4), jnp.float32)], axis=1)
    rbv = jnp.concatenate([jnp.full((4,), rel_bias[0], jnp.float32),
                           jnp.full((4,), rel_bias[1], jnp.float32)]
                          ).reshape(1, 8)

    hbf, hf, auxd, aux2 = _phase1(x, W, b2d, Pmat, jnp.asarray(_ILV))
    accp = _phase2(edge_index, hbf, auxd)
    out = _phase3(accp, aux2, hf, Q, jnp.asarray(_S_SEL), rbv,
                  jnp.asarray(_PERM))
    return out
